# baseline (speedup 1.0000x reference)
import functools

import jax
import jax.numpy as jnp
from jax import lax
from jax.experimental import pallas as pl
from jax.experimental.pallas import tpu as pltpu
from jax.experimental.pallas import tpu_sc as plsc

_D = 32
_NW = 32
_BB = 1024
_H = 200
_B = 4096
_NBT = _B // 128
_UPH = _B // _BB
_NUNIT = _H * _UPH
_PER_W = _NUNIT // _NW
_NBUF = 2
_SLAB = (_BB // 128) * 8 * 128

_mesh = plsc.VectorSubcoreMesh(core_axis_name="c", subcore_axis_name="s")


@functools.partial(
    pl.kernel,
    mesh=_mesh,
    out_type=jax.ShapeDtypeStruct((_H, _D // 8, _NBT * 8 * 128), jnp.float32),
    scratch_types=[
        pltpu.VMEM((_NBUF, _BB), jnp.int32),
        pltpu.VMEM((_NBUF, _BB, _D), jnp.float32),
        pltpu.VMEM((_NBUF, _SLAB), jnp.float32),
        pltpu.SemaphoreType.DMA((_NBUF,)),
        pltpu.SemaphoreType.DMA((_NBUF,)),
        pltpu.SemaphoreType.DMA((_NBUF,)),
    ],
    compiler_params=pltpu.CompilerParams(use_tc_tiling_on_sc=False, needs_layout_passes=False),
)
def _gather_kernel(idx_hbm, table_hbm, out_hbm, idx_v, rows_v, slab_v,
                   sem_i, sem_g, sem_s):
    wid = lax.axis_index("s") * 2 + lax.axis_index("c")
    u0 = wid * _PER_W
    iota16 = lax.iota(jnp.int32, 16)

    def idx_cp(i):
        b = i % _NBUF
        return pltpu.make_async_copy(
            idx_hbm.at[pl.ds((u0 + i) * _BB, _BB)], idx_v.at[b], sem_i.at[b])

    def gath(i):
        b = i % _NBUF
        return pltpu.make_async_copy(
            table_hbm.at[idx_v.at[b]], rows_v.at[b], sem_g.at[b])

    def slab_cp(sb, h, dt, bb):
        return pltpu.make_async_copy(
            slab_v.at[sb],
            out_hbm.at[h, dt, pl.ds(bb * _SLAB, _SLAB)], sem_s.at[sb])

    idx_cp(0).start()
    if _PER_W > 1:
        idx_cp(1).start()

    def unit(i, carry):
        b = i % _NBUF
        idx_cp(i).wait()
        gath(i).start()
        gath(i).wait()

        @pl.when(i + _NBUF < _PER_W)
        def _():
            idx_cp(i + _NBUF).start()

        u = u0 + i
        h = u // _UPH
        bb = u % _UPH
        rows = rows_v.at[b]
        for dt in range(_D // 8):
            sb = dt % _NBUF
            @pl.when(i * 4 + dt >= _NBUF)
            def _():
                slab_cp(sb, 0, 0, 0).wait()

            def btl_body(btl, c):
                rbase = btl * 128
                obase = btl * 1024
                for dr in range(8):
                    col = jnp.full((16,), dt * 8 + dr, jnp.int32)
                    for k in range(8):
                        vec = plsc.load_gather(
                            rows, [iota16 + (rbase + k * 16), col])
                        slab_v[sb, pl.ds(obase + dr * 128 + k * 16, 16)] = vec
                return c

            lax.fori_loop(0, _BB // 128, btl_body, 0)
            slab_cp(sb, h, dt, bb).start()
        return carry

    lax.fori_loop(0, _PER_W, unit, 0)
    for sb in range(_NBUF):
        slab_cp(sb, 0, 0, 0).wait()


def kernel(x, embed):
    xt = jnp.transpose(x).reshape(_H * _B)
    out3 = _gather_kernel(xt, embed)
    out5 = out3.reshape(_H, _D // 8, _NBT, 8, 128)
    return jnp.transpose(out5, (2, 4, 0, 1, 3)).reshape(_B, _H, _D)


# speedup vs baseline: 1.0449x; 1.0449x over previous
import functools

import jax
import jax.numpy as jnp
from jax import lax
from jax.experimental import pallas as pl
from jax.experimental.pallas import tpu as pltpu
from jax.experimental.pallas import tpu_sc as plsc

_D = 32
_NW = 32
_BB = 1024
_H = 200
_B = 4096
_NBT = _B // 128
_UPH = _B // _BB
_NUNIT = _H * _UPH
_PER_W = _NUNIT // _NW
_NBUF = 2
_SLAB = (_BB // 128) * 8 * 128

_mesh = plsc.VectorSubcoreMesh(core_axis_name="c", subcore_axis_name="s")


@functools.partial(
    pl.kernel,
    mesh=_mesh,
    out_type=jax.ShapeDtypeStruct((_H, _D // 8, _NBT * 8 * 128), jnp.float32),
    scratch_types=[
        pltpu.VMEM((2 * _NBUF, _BB), jnp.int32),
        pltpu.VMEM((_NBUF, _BB, _D), jnp.float32),
        pltpu.VMEM((_NBUF, _SLAB), jnp.float32),
        pltpu.SemaphoreType.DMA((2 * _NBUF,)),
        pltpu.SemaphoreType.DMA((_NBUF,)),
        pltpu.SemaphoreType.DMA((_NBUF,)),
    ],
    compiler_params=pltpu.CompilerParams(use_tc_tiling_on_sc=False, needs_layout_passes=False),
)
def _gather_kernel(idx_hbm, table_hbm, out_hbm, idx_v, rows_v, slab_v,
                   sem_i, sem_g, sem_s):
    wid = lax.axis_index("s") * 2 + lax.axis_index("c")
    u0 = wid * _PER_W
    iota16 = lax.iota(jnp.int32, 16)

    def idx_cp(i):
        b = i % (2 * _NBUF)
        return pltpu.make_async_copy(
            idx_hbm.at[pl.ds((u0 + i) * _BB, _BB)], idx_v.at[b], sem_i.at[b])

    def gath(i):
        b = i % _NBUF
        return pltpu.make_async_copy(
            table_hbm.at[idx_v.at[i % (2 * _NBUF)]], rows_v.at[b],
            sem_g.at[b])

    def slab_cp(sb, h, dt, bb):
        return pltpu.make_async_copy(
            slab_v.at[sb],
            out_hbm.at[h, dt, pl.ds(bb * _SLAB, _SLAB)], sem_s.at[sb])

    for j in range(min(2 * _NBUF, _PER_W)):
        idx_cp(j).start()
    idx_cp(0).wait()
    gath(0).start()

    def unit(i, carry):
        b = i % _NBUF
        gath(i).wait()

        @pl.when(i + 1 < _PER_W)
        def _():
            idx_cp(i + 1).wait()
            gath(i + 1).start()

        @pl.when(i + 2 * _NBUF < _PER_W)
        def _():
            idx_cp(i + 2 * _NBUF).start()

        u = u0 + i
        h = u // _UPH
        bb = u % _UPH
        rows = rows_v.at[b]
        for dt in range(_D // 8):
            sb = dt % _NBUF
            @pl.when(i * 4 + dt >= _NBUF)
            def _():
                slab_cp(sb, 0, 0, 0).wait()

            cols = [jnp.full((16,), dt * 8 + dr, jnp.int32) for dr in range(8)]

            def btl_body(btl, c):
                rbase = btl * 128
                obase = btl * 1024
                for k in range(8):
                    rowv = iota16 + (rbase + k * 16)
                    for dr in range(8):
                        vec = plsc.load_gather(rows, [rowv, cols[dr]])
                        slab_v[sb, pl.ds(obase + dr * 128 + k * 16, 16)] = vec
                return c

            lax.fori_loop(0, _BB // 128, btl_body, 0)
            slab_cp(sb, h, dt, bb).start()
        return carry

    lax.fori_loop(0, _PER_W, unit, 0)
    for sb in range(_NBUF):
        slab_cp(sb, 0, 0, 0).wait()


def kernel(x, embed):
    xt = jnp.transpose(x).reshape(_H * _B)
    out3 = _gather_kernel(xt, embed)
    out5 = out3.reshape(_H, _D // 8, _NBT, 8, 128)
    return jnp.transpose(out5, (2, 4, 0, 1, 3)).reshape(_B, _H, _D)


# speedup vs baseline: 1.1997x; 1.1481x over previous
import functools

import jax
import jax.numpy as jnp
from jax import lax
from jax.experimental import pallas as pl
from jax.experimental.pallas import tpu as pltpu
from jax.experimental.pallas import tpu_sc as plsc

_D = 32
_NW = 32
_B = 4096
_H = 200
_NVEC = _B * _H
_CHUNK = 1024
_NBUF = 3
_IRING = 2 * _NBUF

_mesh = plsc.VectorSubcoreMesh(core_axis_name="c", subcore_axis_name="s")


def _make_gather(n_vec):
    v_per_w = n_vec // _NW
    n_chunk = v_per_w // _CHUNK

    @functools.partial(
        pl.kernel,
        mesh=_mesh,
        out_type=jax.ShapeDtypeStruct((n_vec, _D), jnp.float32),
        scratch_types=[
            pltpu.VMEM((_IRING, _CHUNK), jnp.int32),
            pltpu.VMEM((_NBUF, _CHUNK, _D), jnp.float32),
            pltpu.SemaphoreType.DMA((_IRING,)),
            pltpu.SemaphoreType.DMA((_NBUF,)),
            pltpu.SemaphoreType.DMA((_NBUF,)),
        ],
        compiler_params=pltpu.CompilerParams(
            use_tc_tiling_on_sc=False, needs_layout_passes=False),
    )
    def gather_kernel(idx_hbm, table_hbm, out_hbm, idx_v, rows_v,
                      sem_i, sem_g, sem_w):
        wid = lax.axis_index("s") * 2 + lax.axis_index("c")
        base = wid * v_per_w

        def idx_cp(i):
            return pltpu.make_async_copy(
                idx_hbm.at[pl.ds(base + i * _CHUNK, _CHUNK)],
                idx_v.at[i % _IRING], sem_i.at[i % _IRING])

        def gath(i):
            b = i % _NBUF
            return pltpu.make_async_copy(
                table_hbm.at[idx_v.at[i % _IRING]], rows_v.at[b],
                sem_g.at[b])

        def wb(i):
            b = i % _NBUF
            return pltpu.make_async_copy(
                rows_v.at[b],
                out_hbm.at[pl.ds(base + i * _CHUNK, _CHUNK)],
                sem_w.at[b])

        for j in range(min(_IRING, n_chunk)):
            idx_cp(j).start()
        idx_cp(0).wait()
        gath(0).start()

        def step(i, carry):
            gath(i).wait()

            @pl.when(i + 1 < n_chunk)
            def _():
                idx_cp(i + 1).wait()

                @pl.when(i >= _NBUF - 1)
                def _():
                    wb(i - (_NBUF - 1)).wait()
                gath(i + 1).start()

            @pl.when(i + _IRING < n_chunk)
            def _():
                idx_cp(i + _IRING).start()

            wb(i).start()
            return carry

        lax.fori_loop(0, n_chunk, step, 0)
        for j in range(max(0, n_chunk - _NBUF + 1), n_chunk):
            wb(j).wait()

    return gather_kernel


def kernel(x, embed):
    n_vec = x.shape[0] * x.shape[1]
    t32 = jnp.pad(embed, ((0, 0), (0, 96))).reshape(4 * embed.shape[0], _D)
    x4 = (x * 4).reshape(n_vec)
    out = _make_gather(n_vec)(x4, t32)
    return out.reshape(x.shape[0], x.shape[1], _D)


# speedup vs baseline: 1.3460x; 1.1220x over previous
import functools

import jax
import jax.numpy as jnp
from jax import lax
from jax.experimental import pallas as pl
from jax.experimental.pallas import tpu as pltpu
from jax.experimental.pallas import tpu_sc as plsc

_D = 32
_NW = 32
_BB = 1024
_H = 200
_B = 4096
_NBT = _B // 128
_UPH = _B // _BB
_NUNIT = _H * _UPH
_PER_W = _NUNIT // _NW
_NBUF = 2
_SLAB = (_BB // 128) * 8 * 128

_mesh = plsc.VectorSubcoreMesh(core_axis_name="c", subcore_axis_name="s")


@functools.partial(
    pl.kernel,
    mesh=_mesh,
    out_type=jax.ShapeDtypeStruct((_H, _D // 8, _NBT * 8 * 128), jnp.float32),
    scratch_types=[
        pltpu.VMEM((2 * _NBUF, _BB), jnp.int32),
        pltpu.VMEM((_NBUF, _BB, _D), jnp.float32),
        pltpu.VMEM((_NBUF, _SLAB), jnp.float32),
        pltpu.SemaphoreType.DMA((2 * _NBUF,)),
        pltpu.SemaphoreType.DMA((_NBUF,)),
        pltpu.SemaphoreType.DMA((_NBUF,)),
    ],
    compiler_params=pltpu.CompilerParams(
        use_tc_tiling_on_sc=False, needs_layout_passes=False,
        disable_bounds_checks=True),
)
def _gather_kernel(idx_hbm, table_hbm, out_hbm, idx_v, rows_v, slab_v,
                   sem_i, sem_g, sem_s):
    wid = lax.axis_index("s") * 2 + lax.axis_index("c")
    u0 = wid * _PER_W
    iota16 = lax.iota(jnp.int32, 16)

    def idx_cp(i):
        b = i % (2 * _NBUF)
        return pltpu.make_async_copy(
            idx_hbm.at[pl.ds((u0 + i) * _BB, _BB)], idx_v.at[b], sem_i.at[b])

    def gath(i):
        b = i % _NBUF
        return pltpu.make_async_copy(
            table_hbm.at[idx_v.at[i % (2 * _NBUF)]], rows_v.at[b],
            sem_g.at[b])

    def slab_cp(sb, h, dt, bb):
        return pltpu.make_async_copy(
            slab_v.at[sb],
            out_hbm.at[h, dt, pl.ds(bb * _SLAB, _SLAB)], sem_s.at[sb])

    for j in range(min(2 * _NBUF, _PER_W)):
        idx_cp(j).start()
    idx_cp(0).wait()
    gath(0).start()

    def unit(i, carry):
        b = i % _NBUF
        gath(i).wait()

        @pl.when(i + 1 < _PER_W)
        def _():
            idx_cp(i + 1).wait()
            gath(i + 1).start()

        @pl.when(i + 2 * _NBUF < _PER_W)
        def _():
            idx_cp(i + 2 * _NBUF).start()

        u = u0 + i
        h = u // _UPH
        bb = u % _UPH
        rows = rows_v.at[b]
        for dt in range(_D // 8):
            sb = dt % _NBUF

            @pl.when(i * 4 + dt >= _NBUF)
            def _():
                slab_cp(sb, 0, 0, 0).wait()

            cols = [jnp.full((16,), dt * 8 + dr, jnp.int32) for dr in range(8)]

            def btl_body(btl):
                rbase = btl * 128
                obase = btl * 1024
                for k in range(8):
                    rowv = iota16 + (rbase + k * 16)
                    for dr in range(8):
                        vec = plsc.load_gather(rows, [rowv, cols[dr]])
                        slab_v[sb, pl.ds(obase + dr * 128 + k * 16, 16)] = vec

            plsc.parallel_loop(0, _BB // 128)(btl_body)
            slab_cp(sb, h, dt, bb).start()
        return carry

    lax.fori_loop(0, _PER_W, unit, 0)
    for sb in range(_NBUF):
        slab_cp(sb, 0, 0, 0).wait()


def kernel(x, embed):
    t32 = jnp.pad(embed, ((0, 0), (0, 96))).reshape(4 * embed.shape[0], _D)
    xt4 = (jnp.transpose(x) * 4).reshape(_H * _B)
    out3 = _gather_kernel(xt4, t32)
    out5 = out3.reshape(_H, _D // 8, _NBT, 8, 128)
    return jnp.transpose(out5, (2, 4, 0, 1, 3)).reshape(_B, _H, _D)
